# baseline (speedup 1.0000x reference)
import jax
import jax.numpy as jnp
from jax.experimental import pallas as pl
from jax.experimental.pallas import tpu as pltpu

_NOPS = 4
_BM = 1024


def _embed_kernel(*refs):
    ids_refs = refs[:_NOPS]
    embs_ref = refs[_NOPS]
    out_ref = refs[_NOPS + 1]
    e = embs_ref[...]
    norm = jnp.sqrt(jnp.sum(e * e, axis=1, keepdims=True))
    normed = e / jnp.maximum(norm, 1e-12)
    for j in range(_NOPS):
        out_ref[j * _BM : (j + 1) * _BM, :] = jnp.dot(
            ids_refs[j][...], normed, preferred_element_type=jnp.float32
        )


def kernel(ids, embs):
    b, v = ids.shape
    _, d = embs.shape
    rows_per_step = _BM * _NOPS
    in_specs = [
        pl.BlockSpec((_BM, v), lambda i, j=j: (i * _NOPS + j, 0))
        for j in range(_NOPS)
    ]
    in_specs.append(pl.BlockSpec((v, d), lambda i: (0, 0)))
    return pl.pallas_call(
        _embed_kernel,
        grid=(b // rows_per_step,),
        in_specs=in_specs,
        out_specs=pl.BlockSpec((rows_per_step, d), lambda i: (i, 0)),
        out_shape=jax.ShapeDtypeStruct((b, d), jnp.float32),
        compiler_params=pltpu.CompilerParams(
            dimension_semantics=("arbitrary",)
        ),
    )(*([ids] * _NOPS), embs)
